# baseline (speedup 1.0000x reference)
import functools

import jax
import jax.numpy as jnp
from jax import lax
from jax.experimental import pallas as pl
from jax.experimental.pallas import tpu as pltpu
from jax.experimental.pallas import tpu_sc as plsc

_N = 10000
_D = 128
_K = 16

_NC = 2
_NS = 16
_NW = _NC * _NS
_C = 100
_CHUNKS = 200
_NBUF = 4

_SELU_SCALE = 1.0507009873554805
_SELU_ALPHA = 1.6732632423543772


def _tc_body(f_ref, w_ref, b_ref, pool_ref, a_ref, cs_ref):
    f = f_ref[...]
    logits = jnp.dot(f, w_ref[...], preferred_element_type=jnp.float32) + b_ref[...]
    mx = jnp.max(logits, axis=1, keepdims=True)
    e = jnp.exp(logits - mx)
    a = e / jnp.sum(e, axis=1, keepdims=True)
    a_ref[...] = a
    cs = jnp.sum(a, axis=0, keepdims=True)
    cs_ref[...] = cs
    ap = a / cs
    p = lax.dot_general(ap, f, (((0,), (0,)), ((), ())),
                        preferred_element_type=jnp.float32)
    pool_ref[...] = jnp.where(
        p > 0.0, _SELU_SCALE * p, _SELU_SCALE * _SELU_ALPHA * (jnp.exp(p) - 1.0))


_tc_call = pl.pallas_call(
    _tc_body,
    out_shape=[
        jax.ShapeDtypeStruct((_K, _D), jnp.float32),
        jax.ShapeDtypeStruct((_N, _K), jnp.float32),
        jax.ShapeDtypeStruct((1, _K), jnp.float32),
    ],
)


@functools.partial(
    pl.kernel,
    out_type=jax.ShapeDtypeStruct((_NW * 8, _K), jnp.float32),
    mesh=plsc.VectorSubcoreMesh(core_axis_name="c", subcore_axis_name="s"),
    compiler_params=pltpu.CompilerParams(use_tc_tiling_on_sc=False,
                                         needs_layout_passes=False),
    scratch_types=[
        pltpu.VMEM((_CHUNKS, _C), jnp.int32),
        pltpu.VMEM((_CHUNKS, _C), jnp.int32),
        [pltpu.VMEM((_C, 8), jnp.int32) for _ in range(_NBUF)],
        [pltpu.VMEM((_C, 8), jnp.int32) for _ in range(_NBUF)],
        pltpu.VMEM((8, _K), jnp.float32),
        [pltpu.SemaphoreType.DMA for _ in range(_NBUF)],
    ],
)
def _sc_call(a_hbm, src_hbm, dst_hbm, out_hbm,
             idx_s, idx_d, rows_s, rows_d, acc, sems):
    wid = lax.axis_index("s") * _NC + lax.axis_index("c")

    pltpu.sync_copy(src_hbm.at[wid], idx_s)
    pltpu.sync_copy(dst_hbm.at[wid], idx_d)

    lanes = lax.iota(jnp.int32, _K)
    col = lanes & 7
    row0 = lanes >> 3

    def fire(c, b):
        pltpu.async_copy(a_hbm.at[idx_s.at[c]], rows_s[b], sems[b])
        pltpu.async_copy(a_hbm.at[idx_d.at[c]], rows_d[b], sems[b])

    def wait(b):
        pltpu.make_async_copy(a_hbm.at[idx_s.at[0]], rows_s[b], sems[b]).wait()
        pltpu.make_async_copy(a_hbm.at[idx_d.at[0]], rows_d[b], sems[b]).wait()

    def compute(b):
        t0 = jnp.zeros((_K,), jnp.float32)
        t1 = jnp.zeros((_K,), jnp.float32)
        sa = jnp.zeros((_K,), jnp.float32)
        sb = jnp.zeros((_K,), jnp.float32)
        row = row0
        for _ in range(_C // 2):
            ws = plsc.load_gather(rows_s[b], [row, col])
            wd = plsc.load_gather(rows_d[b], [row, col])
            se, so = plsc.unpack(plsc.bitcast(ws, jnp.bfloat16),
                                 format=plsc.PackFormat.INTERLEAVED)
            de, do_ = plsc.unpack(plsc.bitcast(wd, jnp.bfloat16),
                                  format=plsc.PackFormat.INTERLEAVED)
            t0 = t0 + se * de
            t1 = t1 + so * do_
            sa = sa + de
            sb = sb + do_
            row = row + 2
        acc[0, :] += t0 + t1
        acc[1, :] += sa
        acc[2, :] += sb

    z = jnp.zeros((_K,), jnp.float32)
    for r in range(8):
        acc[r, :] = z

    for b in range(_NBUF):
        fire(b, b)

    def group(j, carry):
        for b in range(_NBUF):
            wait(b)
            compute(b)

            @pl.when(j < _CHUNKS // _NBUF - 1)
            def _next():
                fire(_NBUF * (j + 1) + b, b)

        return carry

    lax.fori_loop(0, _CHUNKS // _NBUF, group, 0)
    pltpu.sync_copy(acc, out_hbm.at[pl.ds(wid * 8, 8)])


def _loss_body(p_ref, cs_ref, lam_ref, out_ref):
    p = p_ref[...]
    r = lax.broadcasted_iota(jnp.int32, (_NW * 8, _K), 0)
    isdot = (r % 8) == 0
    isa = (r % 8) == 1
    isb = (r % 8) == 2
    zero = jnp.zeros_like(p)

    m = jnp.float32(_CHUNKS * _C * 16)
    ne_half = jnp.float32(_CHUNKS * _C * 8)

    def term(sel, ne):
        t = jnp.sum(jnp.where(isdot & sel, p, zero))
        sa = jnp.sum(jnp.where(isa & sel, p, zero), axis=0, keepdims=True)
        sb = jnp.sum(jnp.where(isb & sel, p, zero), axis=0, keepdims=True)
        se = sa[:, :8] + sa[:, 8:]
        so = sb[:, :8] + sb[:, 8:]
        ss = jnp.sum(se * se) + jnp.sum(so * so)
        return -(t - ss / (2.0 * ne)) / (2.0 * m)

    adj_loss = term(r < 128, m)
    red_loss = term((r >= 128) & (r < 192), ne_half)
    blue_loss = term(r >= 192, ne_half)

    cs = cs_ref[...]
    collapse_loss = (jnp.sqrt(jnp.sum(cs * cs)) / _N * jnp.sqrt(jnp.float32(_K))
                     - 1.0)
    lam = lam_ref[0, 0]
    lam_f = lam.astype(jnp.float32)
    fair_term = jnp.abs(lam_f * (red_loss - blue_loss))
    total = (jnp.where(lam != 0, fair_term, jnp.float32(0.0))
             + jnp.where(lam != 1, adj_loss, jnp.float32(0.0))
             + jnp.float32(0.1) * collapse_loss)
    out_ref[...] = jnp.full((1, 1), 0.0, jnp.float32) + total


_loss_call = pl.pallas_call(
    _loss_body,
    in_specs=[
        pl.BlockSpec(memory_space=pltpu.VMEM),
        pl.BlockSpec(memory_space=pltpu.VMEM),
        pl.BlockSpec(memory_space=pltpu.SMEM),
    ],
    out_specs=pl.BlockSpec(memory_space=pltpu.VMEM),
    out_shape=jax.ShapeDtypeStruct((1, 1), jnp.float32),
)


def kernel(features, adj_indices, red_indices, blue_indices, W, b, lamda):
    features_pooled, assignments, cs2 = _tc_call(features, W, b.reshape(1, _K))

    src_all = jnp.concatenate(
        [adj_indices[0], red_indices[0], blue_indices[0]]).reshape(_NW, _CHUNKS, _C)
    dst_all = jnp.concatenate(
        [adj_indices[1], red_indices[1], blue_indices[1]]).reshape(_NW, _CHUNKS, _C)

    a_pk = lax.bitcast_convert_type(
        assignments.astype(jnp.bfloat16).reshape(_N, 8, 2), jnp.int32)
    partials = _sc_call(a_pk, src_all, dst_all)

    lam = jnp.asarray(lamda, jnp.int32).reshape(1, 1)
    total_loss = _loss_call(partials, cs2, lam)[0, 0]

    return (features_pooled, assignments, total_loss)


# speedup vs baseline: 1.1350x; 1.1350x over previous
import functools

import jax
import jax.numpy as jnp
from jax import lax
from jax.experimental import pallas as pl
from jax.experimental.pallas import tpu as pltpu
from jax.experimental.pallas import tpu_sc as plsc

_N = 10000
_D = 128
_K = 16

_NC = 2
_NS = 16
_NW = _NC * _NS
_C = 100
_CHUNKS = 200
_NBUF = 4

_SELU_SCALE = 1.0507009873554805
_SELU_ALPHA = 1.6732632423543772


def _tc_body(f_ref, w_ref, b_ref, pool_ref, a_ref, cs_ref, apk_ref):
    f = f_ref[...]
    logits = jnp.dot(f, w_ref[...], preferred_element_type=jnp.float32) + b_ref[...]
    mx = jnp.max(logits, axis=1, keepdims=True)
    e = jnp.exp(logits - mx)
    a = e / jnp.sum(e, axis=1, keepdims=True)
    a_ref[...] = a
    cs = jnp.sum(a, axis=0, keepdims=True)
    cs_ref[...] = cs
    u = lax.bitcast_convert_type(a, jnp.uint32)
    r = (u + 0x7FFF + ((u >> 16) & 1)) >> 16
    apk_ref[...] = lax.bitcast_convert_type(r[:, 0:8] | (r[:, 8:16] << 16),
                                            jnp.int32)
    ap = a / cs
    p = lax.dot_general(ap, f, (((0,), (0,)), ((), ())),
                        preferred_element_type=jnp.float32)
    pool_ref[...] = jnp.where(
        p > 0.0, _SELU_SCALE * p, _SELU_SCALE * _SELU_ALPHA * (jnp.exp(p) - 1.0))


_tc_call = pl.pallas_call(
    _tc_body,
    out_shape=[
        jax.ShapeDtypeStruct((_K, _D), jnp.float32),
        jax.ShapeDtypeStruct((_N, _K), jnp.float32),
        jax.ShapeDtypeStruct((1, _K), jnp.float32),
        jax.ShapeDtypeStruct((_N, 8), jnp.int32),
    ],
)


@functools.partial(
    pl.kernel,
    out_type=jax.ShapeDtypeStruct((_NW * 8, _K), jnp.float32),
    mesh=plsc.VectorSubcoreMesh(core_axis_name="c", subcore_axis_name="s"),
    compiler_params=pltpu.CompilerParams(use_tc_tiling_on_sc=False,
                                         needs_layout_passes=False),
    scratch_types=[
        pltpu.VMEM((_CHUNKS, _C), jnp.int32),
        pltpu.VMEM((_CHUNKS, _C), jnp.int32),
        [pltpu.VMEM((_C, 8), jnp.int32) for _ in range(_NBUF)],
        [pltpu.VMEM((_C, 8), jnp.int32) for _ in range(_NBUF)],
        pltpu.VMEM((8, _K), jnp.float32),
        [pltpu.SemaphoreType.DMA for _ in range(_NBUF)],
    ],
)
def _sc_call(a_hbm, idx_hbm, out_hbm,
             idx_s, idx_d, rows_s, rows_d, acc, sems):
    wid = lax.axis_index("s") * _NC + lax.axis_index("c")

    pltpu.sync_copy(idx_hbm.at[0, wid], idx_s)
    pltpu.sync_copy(idx_hbm.at[1, wid], idx_d)

    lanes = lax.iota(jnp.int32, _K)
    col = lanes & 7
    row0 = lanes >> 3

    def fire(c, b):
        pltpu.async_copy(a_hbm.at[idx_s.at[c]], rows_s[b], sems[b])
        pltpu.async_copy(a_hbm.at[idx_d.at[c]], rows_d[b], sems[b])

    def wait(b):
        pltpu.make_async_copy(a_hbm.at[idx_s.at[0]], rows_s[b], sems[b]).wait()
        pltpu.make_async_copy(a_hbm.at[idx_d.at[0]], rows_d[b], sems[b]).wait()

    def compute(b):
        t0 = jnp.zeros((_K,), jnp.float32)
        t1 = jnp.zeros((_K,), jnp.float32)
        sa = jnp.zeros((_K,), jnp.float32)
        sb = jnp.zeros((_K,), jnp.float32)
        row = row0
        for _ in range(_C // 2):
            ws = plsc.load_gather(rows_s[b], [row, col])
            wd = plsc.load_gather(rows_d[b], [row, col])
            se, so = plsc.unpack(plsc.bitcast(ws, jnp.bfloat16),
                                 format=plsc.PackFormat.INTERLEAVED)
            de, do_ = plsc.unpack(plsc.bitcast(wd, jnp.bfloat16),
                                  format=plsc.PackFormat.INTERLEAVED)
            t0 = t0 + se * de
            t1 = t1 + so * do_
            sa = sa + de
            sb = sb + do_
            row = row + 2
        acc[0, :] += t0 + t1
        acc[1, :] += sa
        acc[2, :] += sb

    z = jnp.zeros((_K,), jnp.float32)
    for r in range(8):
        acc[r, :] = z

    for b in range(_NBUF):
        fire(b, b)

    def group(j, carry):
        for b in range(_NBUF):
            wait(b)
            compute(b)

            @pl.when(j < _CHUNKS // _NBUF - 1)
            def _next():
                fire(_NBUF * (j + 1) + b, b)

        return carry

    lax.fori_loop(0, _CHUNKS // _NBUF, group, 0)
    pltpu.sync_copy(acc, out_hbm.at[pl.ds(wid * 8, 8)])


def _loss_body(p_ref, cs_ref, lam_ref, out_ref):
    p = p_ref[...]
    r = lax.broadcasted_iota(jnp.int32, (_NW * 8, _K), 0)
    isdot = (r % 8) == 0
    isa = (r % 8) == 1
    isb = (r % 8) == 2
    zero = jnp.zeros_like(p)

    m = jnp.float32(_CHUNKS * _C * 16)
    ne_half = jnp.float32(_CHUNKS * _C * 8)

    def term(sel, ne):
        t = jnp.sum(jnp.where(isdot & sel, p, zero))
        sa = jnp.sum(jnp.where(isa & sel, p, zero), axis=0, keepdims=True)
        sb = jnp.sum(jnp.where(isb & sel, p, zero), axis=0, keepdims=True)
        se = sa[:, :8] + sa[:, 8:]
        so = sb[:, :8] + sb[:, 8:]
        ss = jnp.sum(se * se) + jnp.sum(so * so)
        return -(t - ss / (2.0 * ne)) / (2.0 * m)

    adj_loss = term(r < 128, m)
    red_loss = term((r >= 128) & (r < 192), ne_half)
    blue_loss = term(r >= 192, ne_half)

    cs = cs_ref[...]
    collapse_loss = (jnp.sqrt(jnp.sum(cs * cs)) / _N * jnp.sqrt(jnp.float32(_K))
                     - 1.0)
    lam = lam_ref[0, 0]
    lam_f = lam.astype(jnp.float32)
    fair_term = jnp.abs(lam_f * (red_loss - blue_loss))
    total = (jnp.where(lam != 0, fair_term, jnp.float32(0.0))
             + jnp.where(lam != 1, adj_loss, jnp.float32(0.0))
             + jnp.float32(0.1) * collapse_loss)
    out_ref[...] = jnp.full((1, 1), 0.0, jnp.float32) + total


_loss_call = pl.pallas_call(
    _loss_body,
    in_specs=[
        pl.BlockSpec(memory_space=pltpu.VMEM),
        pl.BlockSpec(memory_space=pltpu.VMEM),
        pl.BlockSpec(memory_space=pltpu.SMEM),
    ],
    out_specs=pl.BlockSpec(memory_space=pltpu.VMEM),
    out_shape=jax.ShapeDtypeStruct((1, 1), jnp.float32),
)


def kernel(features, adj_indices, red_indices, blue_indices, W, b, lamda):
    features_pooled, assignments, cs2, a_pk = _tc_call(features, W, b.reshape(1, _K))

    idx_all = jnp.concatenate(
        [adj_indices, red_indices, blue_indices], axis=1).reshape(
            2, _NW, _CHUNKS, _C)

    partials = _sc_call(a_pk, idx_all)

    lam = jnp.asarray(lamda, jnp.int32).reshape(1, 1)
    total_loss = _loss_call(partials, cs2, lam)[0, 0]

    return (features_pooled, assignments, total_loss)


# speedup vs baseline: 1.1864x; 1.0453x over previous
import functools

import jax
import jax.numpy as jnp
from jax import lax
from jax.experimental import pallas as pl
from jax.experimental.pallas import tpu as pltpu
from jax.experimental.pallas import tpu_sc as plsc

_N = 10000
_D = 128
_K = 16

_NC = 2
_NS = 16
_NW = _NC * _NS
_C = 125
_CHUNKS = 160
_NBUF = 4

_SELU_SCALE = 1.0507009873554805
_SELU_ALPHA = 1.6732632423543772


def _tc_body(f_ref, w_ref, b_ref, pool_ref, a_ref, cs_ref):
    f = f_ref[...]
    logits = jnp.dot(f, w_ref[...], preferred_element_type=jnp.float32) + b_ref[...]
    mx = jnp.max(logits, axis=1, keepdims=True)
    e = jnp.exp(logits - mx)
    a = e / jnp.sum(e, axis=1, keepdims=True)
    a_ref[...] = a
    cs = jnp.sum(a, axis=0, keepdims=True)
    cs_ref[...] = cs
    ap = a / cs
    p = lax.dot_general(ap, f, (((0,), (0,)), ((), ())),
                        preferred_element_type=jnp.float32)
    pool_ref[...] = jnp.where(
        p > 0.0, _SELU_SCALE * p, _SELU_SCALE * _SELU_ALPHA * (jnp.exp(p) - 1.0))


_tc_call = pl.pallas_call(
    _tc_body,
    out_shape=[
        jax.ShapeDtypeStruct((_K, _D), jnp.float32),
        jax.ShapeDtypeStruct((_N, _K), jnp.float32),
        jax.ShapeDtypeStruct((1, _K), jnp.float32),
    ],
)


@functools.partial(
    pl.kernel,
    out_type=jax.ShapeDtypeStruct((_NW * 8, _K), jnp.float32),
    mesh=plsc.VectorSubcoreMesh(core_axis_name="c", subcore_axis_name="s"),
    compiler_params=pltpu.CompilerParams(use_tc_tiling_on_sc=False),
    scratch_types=[
        pltpu.VMEM((_CHUNKS, _C), jnp.int32),
        pltpu.VMEM((_CHUNKS, _C), jnp.int32),
        [pltpu.VMEM((_C, _K), jnp.float32) for _ in range(_NBUF)],
        [pltpu.VMEM((_C, _K), jnp.float32) for _ in range(_NBUF)],
        pltpu.VMEM((8, _K), jnp.float32),
        [pltpu.SemaphoreType.DMA for _ in range(_NBUF)],
    ],
)
def _sc_call(a_hbm, idx_hbm, out_hbm,
             idx_s, idx_d, rows_s, rows_d, acc, sems):
    wid = lax.axis_index("s") * _NC + lax.axis_index("c")

    pltpu.sync_copy(idx_hbm.at[0, wid], idx_s)
    pltpu.sync_copy(idx_hbm.at[1, wid], idx_d)
    acc[...] = jnp.zeros((8, _K), jnp.float32)

    def fire(c, b):
        pltpu.async_copy(a_hbm.at[idx_s.at[c]], rows_s[b], sems[b])
        pltpu.async_copy(a_hbm.at[idx_d.at[c]], rows_d[b], sems[b])

    def wait(b):
        pltpu.make_async_copy(a_hbm.at[idx_s.at[0]], rows_s[b], sems[b]).wait()
        pltpu.make_async_copy(a_hbm.at[idx_d.at[0]], rows_d[b], sems[b]).wait()

    def compute(b):
        t = [jnp.zeros((_K,), jnp.float32) for _ in range(4)]
        s = [jnp.zeros((_K,), jnp.float32) for _ in range(4)]
        for i in range(_C):
            rs = rows_s[b][i, :]
            rd = rows_d[b][i, :]
            t[i % 4] = t[i % 4] + rs * rd
            s[i % 4] = s[i % 4] + rd
        acc[0, :] += (t[0] + t[1]) + (t[2] + t[3])
        acc[1, :] += (s[0] + s[1]) + (s[2] + s[3])

    for b in range(_NBUF):
        fire(b, b)

    def group(j, carry):
        for b in range(_NBUF):
            wait(b)
            compute(b)

            @pl.when(j < _CHUNKS // _NBUF - 1)
            def _next():
                fire(_NBUF * (j + 1) + b, b)

        return carry

    lax.fori_loop(0, _CHUNKS // _NBUF, group, 0)
    pltpu.sync_copy(acc, out_hbm.at[pl.ds(wid * 8, 8)])


def _loss_body(p_ref, cs_ref, lam_ref, out_ref):
    p = p_ref[...]
    r = lax.broadcasted_iota(jnp.int32, (_NW * 8, _K), 0)
    isdot = (r % 8) == 0
    iss = (r % 8) == 1
    zero = jnp.zeros_like(p)

    m = jnp.float32(_CHUNKS * _C * 16)
    ne_half = jnp.float32(_CHUNKS * _C * 8)

    def term(sel, ne):
        t = jnp.sum(jnp.where(isdot & sel, p, zero))
        s = jnp.sum(jnp.where(iss & sel, p, zero), axis=0, keepdims=True)
        return -(t - jnp.sum(s * s) / (2.0 * ne)) / (2.0 * m)

    adj_loss = term(r < 128, m)
    red_loss = term((r >= 128) & (r < 192), ne_half)
    blue_loss = term(r >= 192, ne_half)

    cs = cs_ref[...]
    collapse_loss = (jnp.sqrt(jnp.sum(cs * cs)) / _N * jnp.sqrt(jnp.float32(_K))
                     - 1.0)
    lam = lam_ref[0, 0]
    lam_f = lam.astype(jnp.float32)
    fair_term = jnp.abs(lam_f * (red_loss - blue_loss))
    total = (jnp.where(lam != 0, fair_term, jnp.float32(0.0))
             + jnp.where(lam != 1, adj_loss, jnp.float32(0.0))
             + jnp.float32(0.1) * collapse_loss)
    out_ref[...] = jnp.full((1, 1), 0.0, jnp.float32) + total


_loss_call = pl.pallas_call(
    _loss_body,
    in_specs=[
        pl.BlockSpec(memory_space=pltpu.VMEM),
        pl.BlockSpec(memory_space=pltpu.VMEM),
        pl.BlockSpec(memory_space=pltpu.SMEM),
    ],
    out_specs=pl.BlockSpec(memory_space=pltpu.VMEM),
    out_shape=jax.ShapeDtypeStruct((1, 1), jnp.float32),
)


def kernel(features, adj_indices, red_indices, blue_indices, W, b, lamda):
    features_pooled, assignments, cs2 = _tc_call(features, W, b.reshape(1, _K))

    idx_all = jnp.concatenate(
        [adj_indices, red_indices, blue_indices], axis=1).reshape(
            2, _NW, _CHUNKS, _C)

    partials = _sc_call(assignments, idx_all)

    lam = jnp.asarray(lamda, jnp.int32).reshape(1, 1)
    total_loss = _loss_call(partials, cs2, lam)[0, 0]

    return (features_pooled, assignments, total_loss)


# speedup vs baseline: 1.6238x; 1.3687x over previous
import functools

import jax
import jax.numpy as jnp
from jax import lax
from jax.experimental import pallas as pl
from jax.experimental.pallas import tpu as pltpu
from jax.experimental.pallas import tpu_sc as plsc

_N = 10000
_D = 128
_K = 16

_NC = 2
_NS = 16
_NW = _NC * _NS
_EPW = 20000
_C = 80
_CHUNKS = 250
_NBUF = 5

_SELU_SCALE = 1.0507009873554805
_SELU_ALPHA = 1.6732632423543772


def _tc_body(f_ref, w_ref, b_ref, pool_ref, a_ref, cs_ref):
    f = f_ref[...]
    logits = jnp.dot(f, w_ref[...], preferred_element_type=jnp.float32) + b_ref[...]
    mx = jnp.max(logits, axis=1, keepdims=True)
    e = jnp.exp(logits - mx)
    a = e / jnp.sum(e, axis=1, keepdims=True)
    a_ref[...] = a
    cs = jnp.sum(a, axis=0, keepdims=True)
    cs_ref[...] = cs
    ap = a / cs
    p = lax.dot_general(ap, f, (((0,), (0,)), ((), ())),
                        preferred_element_type=jnp.float32)
    pool_ref[...] = jnp.where(
        p > 0.0, _SELU_SCALE * p, _SELU_SCALE * _SELU_ALPHA * (jnp.exp(p) - 1.0))


_tc_call = pl.pallas_call(
    _tc_body,
    out_shape=[
        jax.ShapeDtypeStruct((_K, _D), jnp.float32),
        jax.ShapeDtypeStruct((_N, _K), jnp.float32),
        jax.ShapeDtypeStruct((1, _K), jnp.float32),
    ],
)


@functools.partial(
    pl.kernel,
    out_type=jax.ShapeDtypeStruct((_NW * 8, _K), jnp.float32),
    mesh=plsc.VectorSubcoreMesh(core_axis_name="c", subcore_axis_name="s"),
    compiler_params=pltpu.CompilerParams(use_tc_tiling_on_sc=False),
    scratch_types=[
        pltpu.VMEM((_EPW,), jnp.int32),
        pltpu.VMEM((_EPW,), jnp.int32),
        [pltpu.VMEM((_C, _K), jnp.float32) for _ in range(_NBUF)],
        [pltpu.VMEM((_C, _K), jnp.float32) for _ in range(_NBUF)],
        pltpu.VMEM((8, _K), jnp.float32),
        [pltpu.SemaphoreType.DMA for _ in range(_NBUF)],
    ],
)
def _sc_call(a_hbm, adj_hbm, red_hbm, blue_hbm, out_hbm,
             idx_s, idx_d, rows_s, rows_d, acc, sems):
    wid = lax.axis_index("s") * _NC + lax.axis_index("c")

    @pl.when(wid < 16)
    def _stage_adj():
        pltpu.sync_copy(adj_hbm.at[0, pl.ds(wid * _EPW, _EPW)], idx_s)
        pltpu.sync_copy(adj_hbm.at[1, pl.ds(wid * _EPW, _EPW)], idx_d)

    @pl.when(jnp.logical_and(wid >= 16, wid < 24))
    def _stage_red():
        pltpu.sync_copy(red_hbm.at[0, pl.ds((wid - 16) * _EPW, _EPW)], idx_s)
        pltpu.sync_copy(red_hbm.at[1, pl.ds((wid - 16) * _EPW, _EPW)], idx_d)

    @pl.when(wid >= 24)
    def _stage_blue():
        pltpu.sync_copy(blue_hbm.at[0, pl.ds((wid - 24) * _EPW, _EPW)], idx_s)
        pltpu.sync_copy(blue_hbm.at[1, pl.ds((wid - 24) * _EPW, _EPW)], idx_d)

    acc[...] = jnp.zeros((8, _K), jnp.float32)

    def fire(c, b):
        pltpu.async_copy(a_hbm.at[idx_s.at[pl.ds(c * _C, _C)]], rows_s[b], sems[b])
        pltpu.async_copy(a_hbm.at[idx_d.at[pl.ds(c * _C, _C)]], rows_d[b], sems[b])

    def wait(b):
        pltpu.make_async_copy(a_hbm.at[idx_s.at[pl.ds(0, _C)]], rows_s[b], sems[b]).wait()
        pltpu.make_async_copy(a_hbm.at[idx_d.at[pl.ds(0, _C)]], rows_d[b], sems[b]).wait()

    def compute(b):
        t = [jnp.zeros((_K,), jnp.float32) for _ in range(4)]
        s = [jnp.zeros((_K,), jnp.float32) for _ in range(4)]
        for i in range(_C):
            rs = rows_s[b][i, :]
            rd = rows_d[b][i, :]
            t[i % 4] = t[i % 4] + rs * rd
            s[i % 4] = s[i % 4] + rd
        acc[0, :] += (t[0] + t[1]) + (t[2] + t[3])
        acc[1, :] += (s[0] + s[1]) + (s[2] + s[3])

    for b in range(_NBUF):
        fire(b, b)

    def group(j, carry):
        for b in range(_NBUF):
            wait(b)
            compute(b)

            @pl.when(j < _CHUNKS // _NBUF - 1)
            def _next():
                fire(_NBUF * (j + 1) + b, b)

        return carry

    lax.fori_loop(0, _CHUNKS // _NBUF, group, 0)
    pltpu.sync_copy(acc, out_hbm.at[pl.ds(wid * 8, 8)])


def _loss_body(p_ref, cs_ref, lam_ref, out_ref):
    p = p_ref[...]
    r = lax.broadcasted_iota(jnp.int32, (_NW * 8, _K), 0)
    isdot = (r % 8) == 0
    iss = (r % 8) == 1
    zero = jnp.zeros_like(p)

    m = jnp.float32(_CHUNKS * _C * 16)
    ne_half = jnp.float32(_CHUNKS * _C * 8)

    def term(sel, ne):
        t = jnp.sum(jnp.where(isdot & sel, p, zero))
        s = jnp.sum(jnp.where(iss & sel, p, zero), axis=0, keepdims=True)
        return -(t - jnp.sum(s * s) / (2.0 * ne)) / (2.0 * m)

    adj_loss = term(r < 128, m)
    red_loss = term((r >= 128) & (r < 192), ne_half)
    blue_loss = term(r >= 192, ne_half)

    cs = cs_ref[...]
    collapse_loss = (jnp.sqrt(jnp.sum(cs * cs)) / _N * jnp.sqrt(jnp.float32(_K))
                     - 1.0)
    lam = lam_ref[0, 0]
    lam_f = lam.astype(jnp.float32)
    fair_term = jnp.abs(lam_f * (red_loss - blue_loss))
    total = (jnp.where(lam != 0, fair_term, jnp.float32(0.0))
             + jnp.where(lam != 1, adj_loss, jnp.float32(0.0))
             + jnp.float32(0.1) * collapse_loss)
    out_ref[...] = jnp.full((1, 1), 0.0, jnp.float32) + total


_loss_call = pl.pallas_call(
    _loss_body,
    in_specs=[
        pl.BlockSpec(memory_space=pltpu.VMEM),
        pl.BlockSpec(memory_space=pltpu.VMEM),
        pl.BlockSpec(memory_space=pltpu.SMEM),
    ],
    out_specs=pl.BlockSpec(memory_space=pltpu.VMEM),
    out_shape=jax.ShapeDtypeStruct((1, 1), jnp.float32),
)


def kernel(features, adj_indices, red_indices, blue_indices, W, b, lamda):
    features_pooled, assignments, cs2 = _tc_call(features, W, b.reshape(1, _K))

    partials = _sc_call(assignments, adj_indices, red_indices, blue_indices)

    lam = jnp.asarray(lamda, jnp.int32).reshape(1, 1)
    total_loss = _loss_call(partials, cs2, lam)[0, 0]

    return (features_pooled, assignments, total_loss)
